# Initial kernel scaffold; baseline (speedup 1.0000x reference)
import jax
import jax.numpy as jnp
from jax.experimental import pallas as pl


def kernel(x, edge_index, edge_attr, W_enc, b_enc, W_ein, b_ein, W_eh, b_eh, W1_0, b1_0, W2_0, b2_0, g_0, be_0, W1_1, b1_1, W2_1, b2_1, g_1, be_1, W1_2, b1_2, W2_2, b2_2, g_2, be_2, Wm1, bm1, Wm2, bm2):
    raise NotImplementedError("write your pallas kernel here")


# speedup vs baseline: 1.4000x; 1.4000x over previous
import functools

import jax
import jax.numpy as jnp
from jax import lax
from jax.experimental import pallas as pl
from jax.experimental.pallas import tpu as pltpu
from jax.experimental.pallas import tpu_sc as plsc

F32 = jnp.float32

NC = 2
NS = 16
NW = NC * NS
LANES = 16
CHUNK = 128


def _cdiv(a, b):
    return (a + b - 1) // b


def _linear_body(x_ref, w_ref, b_ref, o_ref, *, act):
    y = jnp.dot(x_ref[...], w_ref[...], preferred_element_type=F32) + b_ref[...]
    if act:
        y = jnp.maximum(y, 0.0)
    o_ref[...] = y


def _tc_linear(x, w, b2d, act=False, block_r=None):
    R, K = x.shape
    Nn = w.shape[1]
    if block_r is None:
        block_r = 400 if R % 400 == 0 else 2048
    grid = (R // block_r,)
    return pl.pallas_call(
        functools.partial(_linear_body, act=act),
        grid=grid,
        in_specs=[
            pl.BlockSpec((block_r, K), lambda i: (i, 0)),
            pl.BlockSpec((K, Nn), lambda i: (0, 0)),
            pl.BlockSpec((1, Nn), lambda i: (0, 0)),
        ],
        out_specs=pl.BlockSpec((block_r, Nn), lambda i: (i, 0)),
        out_shape=jax.ShapeDtypeStruct((R, Nn), F32),
    )(x, w, b2d)


def _edge_proj_body(ea_ref, wi_ref, bi_ref, wh_ref, bh_ref, o0_ref, oh_ref):
    ea = ea_ref[...]
    o0_ref[...] = jnp.dot(ea, wi_ref[...], preferred_element_type=F32) + bi_ref[...]
    oh_ref[...] = jnp.dot(ea, wh_ref[...], preferred_element_type=F32) + bh_ref[...]


def _tc_edge_proj(ea, w_in, b_in, w_h, b_h):
    Ep, ED = ea.shape
    D = w_in.shape[1]
    block_r = 2048
    grid = (Ep // block_r,)
    return pl.pallas_call(
        _edge_proj_body,
        grid=grid,
        in_specs=[
            pl.BlockSpec((block_r, ED), lambda i: (i, 0)),
            pl.BlockSpec((ED, D), lambda i: (0, 0)),
            pl.BlockSpec((1, D), lambda i: (0, 0)),
            pl.BlockSpec((ED, D), lambda i: (0, 0)),
            pl.BlockSpec((1, D), lambda i: (0, 0)),
        ],
        out_specs=[
            pl.BlockSpec((block_r, D), lambda i: (i, 0)),
            pl.BlockSpec((block_r, D), lambda i: (i, 0)),
        ],
        out_shape=[
            jax.ShapeDtypeStruct((Ep, D), F32),
            jax.ShapeDtypeStruct((Ep, D), F32),
        ],
    )(ea, w_in, b_in, w_h, b_h)


def _mlp_body(h_ref, a_ref, w1_ref, b1_ref, w2_ref, b2_ref, sc_ref, be_ref, o_ref):
    z = h_ref[...] + a_ref[0] + a_ref[1]
    t = jnp.maximum(jnp.dot(z, w1_ref[...], preferred_element_type=F32) + b1_ref[...], 0.0)
    u = jnp.dot(t, w2_ref[...], preferred_element_type=F32) + b2_ref[...]
    o_ref[...] = jnp.maximum(u * sc_ref[...] + be_ref[...], 0.0)


def _tc_layer_mlp(h, aggr2, w1, b1, w2, b2, scale, be):
    R, D = h.shape
    H = w1.shape[1]
    block_r = 400
    grid = (R // block_r,)
    return pl.pallas_call(
        _mlp_body,
        grid=grid,
        in_specs=[
            pl.BlockSpec((block_r, D), lambda i: (i, 0)),
            pl.BlockSpec((2, block_r, D), lambda i: (0, i, 0)),
            pl.BlockSpec((D, H), lambda i: (0, 0)),
            pl.BlockSpec((1, H), lambda i: (0, 0)),
            pl.BlockSpec((H, H), lambda i: (0, 0)),
            pl.BlockSpec((1, H), lambda i: (0, 0)),
            pl.BlockSpec((1, H), lambda i: (0, 0)),
            pl.BlockSpec((1, H), lambda i: (0, 0)),
        ],
        out_specs=pl.BlockSpec((block_r, H), lambda i: (i, 0)),
        out_shape=jax.ShapeDtypeStruct((R, H), F32),
    )(h, aggr2, w1, b1, w2, b2, scale, be)


def _ab_body(h_ref, wt_ref, bt_ref, wb_ref, a_ref, b_ref):
    h = h_ref[...]
    a_ref[...] = jnp.dot(h, wt_ref[...], preferred_element_type=F32) + bt_ref[...]
    b_ref[...] = jnp.dot(h, wb_ref[...], preferred_element_type=F32)


def _tc_head_ab(h, w_top, bm1, w_bot):
    R, H = h.shape
    block_r = 400
    grid = (R // block_r,)
    return pl.pallas_call(
        _ab_body,
        grid=grid,
        in_specs=[
            pl.BlockSpec((block_r, H), lambda i: (i, 0)),
            pl.BlockSpec((H, H), lambda i: (0, 0)),
            pl.BlockSpec((1, H), lambda i: (0, 0)),
            pl.BlockSpec((H, H), lambda i: (0, 0)),
        ],
        out_specs=[
            pl.BlockSpec((block_r, H), lambda i: (i, 0)),
            pl.BlockSpec((block_r, H), lambda i: (i, 0)),
        ],
        out_shape=[
            jax.ShapeDtypeStruct((R, H), F32),
            jax.ShapeDtypeStruct((R, H), F32),
        ],
    )(h, w_top, bm1, w_bot)


@functools.lru_cache(maxsize=None)
def _make_msg_kernel(n_nodes, e_pad):
    per_w = e_pad // NW
    n_chunks = per_w // CHUNK
    n_acc = _cdiv(n_nodes + 1, NS * 8) * NS * 8
    rows_t = n_acc // NS
    mesh = plsc.VectorSubcoreMesh(core_axis_name="c", subcore_axis_name="s",
                                  num_cores=NC, num_subcores=NS)

    def _chunk_sizes(total):
        sizes = []
        while total > 0:
            sizes.append(min(CHUNK, total))
            total -= sizes[-1]
        return sizes

    @functools.partial(
        pl.kernel,
        out_type=jax.ShapeDtypeStruct((NC, n_acc, 128), F32),
        mesh=mesh,
        scratch_types=[
            pltpu.VMEM((CHUNK,), jnp.int32),
            pltpu.VMEM((CHUNK,), jnp.int32),
            pltpu.VMEM((CHUNK, 128), F32),
            pltpu.VMEM((CHUNK, 128), F32),
            pltpu.VMEM((CHUNK, 128), F32),
            pltpu.VMEM_SHARED((n_acc, 128), F32),
            pltpu.SemaphoreType.DMA,
        ],
    )
    def msg_kernel(h_hbm, ee_hbm, src_hbm, dst_hbm, out_hbm,
                   sidx, didx, hbuf, eebuf, zbuf, acc, sem):
        c = lax.axis_index("c")
        s = lax.axis_index("s")
        wid = s * NC + c
        zero16 = jnp.zeros((LANES,), F32)

        def zrow(i, carry):
            for j in range(8):
                zbuf[i, pl.ds(j * LANES, LANES)] = zero16
            return carry

        lax.fori_loop(0, CHUNK, zrow, 0)
        zbase = pl.multiple_of(s * rows_t, 8)
        off = 0
        for sz in _chunk_sizes(rows_t):
            pltpu.sync_copy(zbuf.at[pl.ds(0, sz)], acc.at[pl.ds(zbase + off, sz)])
            off += sz
        plsc.subcore_barrier()

        ebase = wid * per_w

        def chunk(k, carry):
            b = pl.multiple_of(ebase + k * CHUNK, 8)
            pltpu.sync_copy(src_hbm.at[pl.ds(b, CHUNK)], sidx)
            pltpu.sync_copy(dst_hbm.at[pl.ds(b, CHUNK)], didx)
            pltpu.sync_copy(ee_hbm.at[pl.ds(b, CHUNK)], eebuf)
            pltpu.async_copy(h_hbm.at[sidx], hbuf, sem).wait()

            def row(i, rcarry):
                for j in range(8):
                    sl = pl.ds(j * LANES, LANES)
                    hbuf[i, sl] = jnp.maximum(hbuf[i, sl] + eebuf[i, sl], 0.0)
                return rcarry

            lax.fori_loop(0, CHUNK, row, 0)
            pltpu.sync_copy(hbuf, acc.at[didx], add=True)
            return carry

        lax.fori_loop(0, n_chunks, chunk, 0)
        plsc.subcore_barrier()

        obase = pl.multiple_of(s * rows_t, 8)
        off = 0
        for sz in _chunk_sizes(rows_t):
            pltpu.sync_copy(acc.at[pl.ds(obase + off, sz)],
                            out_hbm.at[c, pl.ds(obase + off, sz)])
            off += sz

    return msg_kernel


@functools.lru_cache(maxsize=None)
def _make_edge_head_kernel(e_pad):
    per_w = e_pad // NW
    n_chunks = per_w // CHUNK
    mesh = plsc.VectorSubcoreMesh(core_axis_name="c", subcore_axis_name="s",
                                  num_cores=NC, num_subcores=NS)

    @functools.partial(
        pl.kernel,
        out_type=jax.ShapeDtypeStruct((e_pad, 128), F32),
        mesh=mesh,
        scratch_types=[
            pltpu.VMEM((CHUNK,), jnp.int32),
            pltpu.VMEM((CHUNK,), jnp.int32),
            pltpu.VMEM((CHUNK, 128), F32),
            pltpu.VMEM((CHUNK, 128), F32),
            pltpu.SemaphoreType.DMA,
        ],
    )
    def edge_head(a_hbm, b_hbm, src_hbm, dst_hbm, m_hbm,
                  sidx, didx, abuf, bbuf, sem):
        c = lax.axis_index("c")
        s = lax.axis_index("s")
        wid = s * NC + c
        ebase = wid * per_w

        def chunk(k, carry):
            b = pl.multiple_of(ebase + k * CHUNK, 8)
            pltpu.sync_copy(src_hbm.at[pl.ds(b, CHUNK)], sidx)
            pltpu.sync_copy(dst_hbm.at[pl.ds(b, CHUNK)], didx)
            pltpu.async_copy(a_hbm.at[sidx], abuf, sem).wait()
            pltpu.async_copy(b_hbm.at[didx], bbuf, sem).wait()

            def row(i, rcarry):
                for j in range(8):
                    sl = pl.ds(j * LANES, LANES)
                    abuf[i, sl] = jnp.maximum(abuf[i, sl] + bbuf[i, sl], 0.0)
                return rcarry

            lax.fori_loop(0, CHUNK, row, 0)
            pltpu.sync_copy(abuf, m_hbm.at[pl.ds(b, CHUNK)])
            return carry

        lax.fori_loop(0, n_chunks, chunk, 0)

    return edge_head


def kernel(x, edge_index, edge_attr, W_enc, b_enc, W_ein, b_ein, W_eh, b_eh,
           W1_0, b1_0, W2_0, b2_0, g_0, be_0, W1_1, b1_1, W2_1, b2_1, g_1, be_1,
           W1_2, b1_2, W2_2, b2_2, g_2, be_2, Wm1, bm1, Wm2, bm2):
    N, D = x.shape
    E = edge_attr.shape[0]
    Ep = _cdiv(E, NW * CHUNK) * (NW * CHUNK)

    src = jnp.pad(edge_index[0], (0, Ep - E))
    dst = jnp.pad(edge_index[1], (0, Ep - E), constant_values=N)
    ea = jnp.pad(edge_attr, ((0, Ep - E), (0, 0)))

    r2 = lambda v: v.reshape(1, -1)
    inv_bn = 1.0 / jnp.sqrt(jnp.float32(1.0 + 1e-5))

    h = _tc_linear(x, W_enc, r2(b_enc))
    ee0, eeh = _tc_edge_proj(ea, W_ein, r2(b_ein), W_eh, r2(b_eh))

    msg = _make_msg_kernel(N, Ep)
    layers = ((W1_0, b1_0, W2_0, b2_0, g_0, be_0),
              (W1_1, b1_1, W2_1, b2_1, g_1, be_1),
              (W1_2, b1_2, W2_2, b2_2, g_2, be_2))
    for li, (W1, b1, W2, b2, g, be) in enumerate(layers):
        ee = ee0 if li == 0 else eeh
        aggr2 = msg(h, ee, src, dst)
        h = _tc_layer_mlp(h, aggr2, W1, r2(b1), W2, r2(b2),
                          r2(g * inv_bn), r2(be))

    A, B = _tc_head_ab(h, Wm1[:D], r2(bm1), Wm1[D:])
    m = _make_edge_head_kernel(Ep)(A, B, src, dst)
    w2p = jnp.pad(Wm2, ((0, 0), (0, 8 - Wm2.shape[1])))
    b2p = jnp.pad(bm2, (0, 8 - bm2.shape[0]))
    out8 = _tc_linear(m, w2p, r2(b2p), block_r=2048)
    return out8[:E, :Wm2.shape[1]]


# speedup vs baseline: 1.9776x; 1.4125x over previous
import functools

import jax
import jax.numpy as jnp
from jax import lax
from jax.experimental import pallas as pl
from jax.experimental.pallas import tpu as pltpu
from jax.experimental.pallas import tpu_sc as plsc

F32 = jnp.float32

NC = 2
NS = 16
NW = NC * NS
LANES = 16
MSG_CHUNK = 40
HEAD_CHUNK = 80
NBUF = 4


def _cdiv(a, b):
    return (a + b - 1) // b


def _linear_body(x_ref, w_ref, b_ref, o_ref, *, act):
    y = jnp.dot(x_ref[...], w_ref[...], preferred_element_type=F32) + b_ref[...]
    if act:
        y = jnp.maximum(y, 0.0)
    o_ref[...] = y


def _tc_linear(x, w, b2d, act=False, block_r=None):
    R, K = x.shape
    Nn = w.shape[1]
    if block_r is None:
        block_r = 400 if R % 400 == 0 else 2048
    grid = (R // block_r,)
    return pl.pallas_call(
        functools.partial(_linear_body, act=act),
        grid=grid,
        in_specs=[
            pl.BlockSpec((block_r, K), lambda i: (i, 0)),
            pl.BlockSpec((K, Nn), lambda i: (0, 0)),
            pl.BlockSpec((1, Nn), lambda i: (0, 0)),
        ],
        out_specs=pl.BlockSpec((block_r, Nn), lambda i: (i, 0)),
        out_shape=jax.ShapeDtypeStruct((R, Nn), F32),
    )(x, w, b2d)


def _edge_proj_body(ea_ref, wi_ref, bi_ref, wh_ref, bh_ref, o0_ref, oh_ref):
    ea = ea_ref[...]
    o0_ref[...] = jnp.dot(ea, wi_ref[...], preferred_element_type=F32) + bi_ref[...]
    oh_ref[...] = jnp.dot(ea, wh_ref[...], preferred_element_type=F32) + bh_ref[...]


def _tc_edge_proj(ea, w_in, b_in, w_h, b_h):
    Ep, ED = ea.shape
    D = w_in.shape[1]
    block_r = 2048
    grid = (Ep // block_r,)
    return pl.pallas_call(
        _edge_proj_body,
        grid=grid,
        in_specs=[
            pl.BlockSpec((block_r, ED), lambda i: (i, 0)),
            pl.BlockSpec((ED, D), lambda i: (0, 0)),
            pl.BlockSpec((1, D), lambda i: (0, 0)),
            pl.BlockSpec((ED, D), lambda i: (0, 0)),
            pl.BlockSpec((1, D), lambda i: (0, 0)),
        ],
        out_specs=[
            pl.BlockSpec((block_r, D), lambda i: (i, 0)),
            pl.BlockSpec((block_r, D), lambda i: (i, 0)),
        ],
        out_shape=[
            jax.ShapeDtypeStruct((Ep, D), F32),
            jax.ShapeDtypeStruct((Ep, D), F32),
        ],
    )(ea, w_in, b_in, w_h, b_h)


def _mlp_body(h_ref, a_ref, w1_ref, b1_ref, w2_ref, b2_ref, sc_ref, be_ref, o_ref):
    z = h_ref[...] + a_ref[0] + a_ref[1]
    t = jnp.maximum(jnp.dot(z, w1_ref[...], preferred_element_type=F32) + b1_ref[...], 0.0)
    u = jnp.dot(t, w2_ref[...], preferred_element_type=F32) + b2_ref[...]
    o_ref[...] = jnp.maximum(u * sc_ref[...] + be_ref[...], 0.0)


def _tc_layer_mlp(h, aggr2, w1, b1, w2, b2, scale, be):
    R, D = h.shape
    H = w1.shape[1]
    block_r = 400
    grid = (R // block_r,)
    return pl.pallas_call(
        _mlp_body,
        grid=grid,
        in_specs=[
            pl.BlockSpec((block_r, D), lambda i: (i, 0)),
            pl.BlockSpec((2, block_r, D), lambda i: (0, i, 0)),
            pl.BlockSpec((D, H), lambda i: (0, 0)),
            pl.BlockSpec((1, H), lambda i: (0, 0)),
            pl.BlockSpec((H, H), lambda i: (0, 0)),
            pl.BlockSpec((1, H), lambda i: (0, 0)),
            pl.BlockSpec((1, H), lambda i: (0, 0)),
            pl.BlockSpec((1, H), lambda i: (0, 0)),
        ],
        out_specs=pl.BlockSpec((block_r, H), lambda i: (i, 0)),
        out_shape=jax.ShapeDtypeStruct((R, H), F32),
    )(h, aggr2, w1, b1, w2, b2, scale, be)


def _ab_body(h_ref, wt_ref, bt_ref, wb_ref, a_ref, b_ref):
    h = h_ref[...]
    a_ref[...] = jnp.dot(h, wt_ref[...], preferred_element_type=F32) + bt_ref[...]
    b_ref[...] = jnp.dot(h, wb_ref[...], preferred_element_type=F32)


def _tc_head_ab(h, w_top, bm1, w_bot):
    R, H = h.shape
    block_r = 400
    grid = (R // block_r,)
    return pl.pallas_call(
        _ab_body,
        grid=grid,
        in_specs=[
            pl.BlockSpec((block_r, H), lambda i: (i, 0)),
            pl.BlockSpec((H, H), lambda i: (0, 0)),
            pl.BlockSpec((1, H), lambda i: (0, 0)),
            pl.BlockSpec((H, H), lambda i: (0, 0)),
        ],
        out_specs=[
            pl.BlockSpec((block_r, H), lambda i: (i, 0)),
            pl.BlockSpec((block_r, H), lambda i: (i, 0)),
        ],
        out_shape=[
            jax.ShapeDtypeStruct((R, H), F32),
            jax.ShapeDtypeStruct((R, H), F32),
        ],
    )(h, w_top, bm1, w_bot)


def _chunk_sizes(total, step):
    sizes = []
    while total > 0:
        sizes.append(min(step, total))
        total -= sizes[-1]
    return sizes


@functools.lru_cache(maxsize=None)
def _make_msg_kernel(n_nodes, e_pad):
    per_w = e_pad // NW
    n_chunks = per_w // MSG_CHUNK
    n_outer = n_chunks // NBUF
    n_acc = _cdiv(n_nodes + 1, NS * 8) * NS * 8
    rows_t = n_acc // NS
    mesh = plsc.VectorSubcoreMesh(core_axis_name="c", subcore_axis_name="s",
                                  num_cores=NC, num_subcores=NS)

    @functools.partial(
        pl.kernel,
        out_type=jax.ShapeDtypeStruct((NC, n_acc, 128), F32),
        mesh=mesh,
        scratch_types=[
            pltpu.VMEM((NBUF, MSG_CHUNK), jnp.int32),
            pltpu.VMEM((NBUF, MSG_CHUNK), jnp.int32),
            pltpu.VMEM((NBUF, MSG_CHUNK, 128), F32),
            pltpu.VMEM((NBUF, MSG_CHUNK, 128), F32),
            pltpu.VMEM((MSG_CHUNK, 128), F32),
            pltpu.VMEM_SHARED((n_acc, 128), F32),
            pltpu.SemaphoreType.DMA,
            pltpu.SemaphoreType.DMA,
            pltpu.SemaphoreType.DMA,
            pltpu.SemaphoreType.DMA,
            pltpu.SemaphoreType.DMA,
            pltpu.SemaphoreType.DMA,
            pltpu.SemaphoreType.DMA,
        ],
    )
    def msg_kernel(h_hbm, ee_hbm, src_hbm, dst_hbm, out_hbm,
                   sidx, didx, hbuf, eebuf, zbuf, acc,
                   sem_i, sem_g0, sem_g1, sem_e0, sem_e1, sem_s0, sem_s1):
        c = lax.axis_index("c")
        s = lax.axis_index("s")
        wid = s * NC + c
        sem_g = (sem_g0, sem_g1)
        sem_e = (sem_e0, sem_e1)
        sem_s = (sem_s0, sem_s1)
        zero16 = jnp.zeros((LANES,), F32)

        def zrow(i, carry):
            for j in range(8):
                zbuf[i, pl.ds(j * LANES, LANES)] = zero16
            return carry

        lax.fori_loop(0, MSG_CHUNK, zrow, 0)
        zbase = pl.multiple_of(s * rows_t, 8)
        off = 0
        for sz in _chunk_sizes(rows_t, MSG_CHUNK):
            pltpu.sync_copy(zbuf.at[pl.ds(0, sz)], acc.at[pl.ds(zbase + off, sz)])
            off += sz
        plsc.subcore_barrier()

        ebase = wid * per_w

        def fire_loads(g, sl):
            b = pl.multiple_of(ebase + g * MSG_CHUNK, 8)
            pltpu.async_copy(src_hbm.at[pl.ds(b, MSG_CHUNK)], sidx.at[sl], sem_i)
            pltpu.async_copy(dst_hbm.at[pl.ds(b, MSG_CHUNK)], didx.at[sl], sem_i)
            pltpu.async_copy(ee_hbm.at[pl.ds(b, MSG_CHUNK)], eebuf.at[sl],
                             sem_e[sl % 2])

        def wait_idx(sl):
            pltpu.make_async_copy(src_hbm.at[pl.ds(0, MSG_CHUNK)], sidx.at[sl],
                                  sem_i).wait()
            pltpu.make_async_copy(dst_hbm.at[pl.ds(0, MSG_CHUNK)], didx.at[sl],
                                  sem_i).wait()

        def fire_gather(sl):
            pltpu.async_copy(h_hbm.at[sidx.at[sl]], hbuf.at[sl], sem_g[sl % 2])

        def wait_gather(sl):
            pltpu.make_async_copy(h_hbm.at[sidx.at[sl]], hbuf.at[sl],
                                  sem_g[sl % 2]).wait()

        def wait_ee(sl):
            pltpu.make_async_copy(ee_hbm.at[pl.ds(0, MSG_CHUNK)], eebuf.at[sl],
                                  sem_e[sl % 2]).wait()

        def fire_scatter(sl):
            pltpu.async_copy(hbuf.at[sl], acc.at[didx.at[sl]], sem_s[sl % 2],
                             add=True)

        def wait_scatter(sl):
            pltpu.make_async_copy(hbuf.at[sl], acc.at[didx.at[sl]],
                                  sem_s[sl % 2]).wait()

        def compute(sl):
            def row(i, rcarry):
                for j in range(8):
                    ds16 = pl.ds(j * LANES, LANES)
                    hbuf[sl, i, ds16] = jnp.maximum(
                        hbuf[sl, i, ds16] + eebuf[sl, i, ds16], 0.0)
                return rcarry

            lax.fori_loop(0, MSG_CHUNK, row, 0)

        fire_loads(0, 0)
        wait_idx(0)
        fire_gather(0)
        fire_loads(1, 1)

        def outer(o, carry):
            for j in range(NBUF):
                g = o * NBUF + j
                s1 = (j + 1) % NBUF
                s2 = (j + 2) % NBUF

                def gather_next(sl=s1):
                    wait_idx(sl)
                    fire_gather(sl)

                if j < NBUF - 1:
                    gather_next()
                else:
                    pl.when(o < n_outer - 1)(gather_next)

                wait_gather(j)
                wait_ee(j)
                compute(j)

                def drain(sl=s2):
                    wait_scatter(sl)

                if j >= 2:
                    drain()
                else:
                    pl.when(o >= 1)(drain)
                fire_scatter(j)

                def load_next(sl=s2, gg=g + 2):
                    fire_loads(gg, sl)

                if j < NBUF - 2:
                    load_next()
                else:
                    pl.when(o < n_outer - 1)(load_next)
            return carry

        lax.fori_loop(0, n_outer, outer, 0)
        wait_scatter(NBUF - 2)
        wait_scatter(NBUF - 1)
        plsc.subcore_barrier()

        obase = pl.multiple_of(s * rows_t, 8)
        off = 0
        for sz in _chunk_sizes(rows_t, MSG_CHUNK):
            pltpu.sync_copy(acc.at[pl.ds(obase + off, sz)],
                            out_hbm.at[c, pl.ds(obase + off, sz)])
            off += sz

    return msg_kernel


@functools.lru_cache(maxsize=None)
def _make_edge_head_kernel(e_pad):
    per_w = e_pad // NW
    n_chunks = per_w // HEAD_CHUNK
    n_outer = n_chunks // NBUF
    mesh = plsc.VectorSubcoreMesh(core_axis_name="c", subcore_axis_name="s",
                                  num_cores=NC, num_subcores=NS)

    @functools.partial(
        pl.kernel,
        out_type=jax.ShapeDtypeStruct((e_pad, 128), F32),
        mesh=mesh,
        scratch_types=[
            pltpu.VMEM((NBUF, HEAD_CHUNK), jnp.int32),
            pltpu.VMEM((NBUF, HEAD_CHUNK), jnp.int32),
            pltpu.VMEM((NBUF, HEAD_CHUNK, 128), F32),
            pltpu.VMEM((NBUF, HEAD_CHUNK, 128), F32),
            pltpu.SemaphoreType.DMA,
            pltpu.SemaphoreType.DMA,
            pltpu.SemaphoreType.DMA,
            pltpu.SemaphoreType.DMA,
            pltpu.SemaphoreType.DMA,
            pltpu.SemaphoreType.DMA,
            pltpu.SemaphoreType.DMA,
        ],
    )
    def edge_head(a_hbm, b_hbm, src_hbm, dst_hbm, m_hbm,
                  sidx, didx, abuf, bbuf,
                  sem_i, sem_a0, sem_a1, sem_b0, sem_b1, sem_w0, sem_w1):
        c = lax.axis_index("c")
        s = lax.axis_index("s")
        wid = s * NC + c
        ebase = wid * per_w
        sem_a = (sem_a0, sem_a1)
        sem_b = (sem_b0, sem_b1)
        sem_w = (sem_w0, sem_w1)

        def fire_loads(g, sl):
            b = pl.multiple_of(ebase + g * HEAD_CHUNK, 8)
            pltpu.async_copy(src_hbm.at[pl.ds(b, HEAD_CHUNK)], sidx.at[sl], sem_i)
            pltpu.async_copy(dst_hbm.at[pl.ds(b, HEAD_CHUNK)], didx.at[sl], sem_i)

        def wait_idx(sl):
            pltpu.make_async_copy(src_hbm.at[pl.ds(0, HEAD_CHUNK)], sidx.at[sl],
                                  sem_i).wait()
            pltpu.make_async_copy(dst_hbm.at[pl.ds(0, HEAD_CHUNK)], didx.at[sl],
                                  sem_i).wait()

        def fire_gathers(sl):
            pltpu.async_copy(a_hbm.at[sidx.at[sl]], abuf.at[sl], sem_a[sl % 2])
            pltpu.async_copy(b_hbm.at[didx.at[sl]], bbuf.at[sl], sem_b[sl % 2])

        def wait_gathers(sl):
            pltpu.make_async_copy(a_hbm.at[sidx.at[sl]], abuf.at[sl],
                                  sem_a[sl % 2]).wait()
            pltpu.make_async_copy(b_hbm.at[didx.at[sl]], bbuf.at[sl],
                                  sem_b[sl % 2]).wait()

        def fire_write(g, sl):
            b = pl.multiple_of(ebase + g * HEAD_CHUNK, 8)
            pltpu.sync_copy(abuf.at[sl], m_hbm.at[pl.ds(b, HEAD_CHUNK)])

        def wait_write(sl):
            del sl

        def compute(sl):
            def row(i, rcarry):
                for j in range(8):
                    ds16 = pl.ds(j * LANES, LANES)
                    abuf[sl, i, ds16] = jnp.maximum(
                        abuf[sl, i, ds16] + bbuf[sl, i, ds16], 0.0)
                return rcarry

            lax.fori_loop(0, HEAD_CHUNK, row, 0)

        fire_loads(0, 0)
        wait_idx(0)
        fire_gathers(0)
        fire_loads(1, 1)

        def outer(o, carry):
            for j in range(NBUF):
                g = o * NBUF + j
                s1 = (j + 1) % NBUF
                s2 = (j + 2) % NBUF

                def gather_next(sl=s1):
                    wait_idx(sl)
                    fire_gathers(sl)

                if j < NBUF - 1:
                    gather_next()
                else:
                    pl.when(o < n_outer - 1)(gather_next)

                wait_gathers(j)
                compute(j)

                def drain(sl=s2):
                    wait_write(sl)

                if j >= 2:
                    drain()
                else:
                    pl.when(o >= 1)(drain)
                fire_write(g, j)

                def load_next(sl=s2, gg=g + 2):
                    fire_loads(gg, sl)

                if j < NBUF - 2:
                    load_next()
                else:
                    pl.when(o < n_outer - 1)(load_next)
            return carry

        lax.fori_loop(0, n_outer, outer, 0)
        wait_write(NBUF - 2)
        wait_write(NBUF - 1)

    return edge_head


def kernel(x, edge_index, edge_attr, W_enc, b_enc, W_ein, b_ein, W_eh, b_eh,
           W1_0, b1_0, W2_0, b2_0, g_0, be_0, W1_1, b1_1, W2_1, b2_1, g_1, be_1,
           W1_2, b1_2, W2_2, b2_2, g_2, be_2, Wm1, bm1, Wm2, bm2):
    N, D = x.shape
    E = edge_attr.shape[0]
    Ep = _cdiv(E, NW * HEAD_CHUNK * NBUF) * (NW * HEAD_CHUNK * NBUF)

    src = jnp.pad(edge_index[0], (0, Ep - E))
    dst = jnp.pad(edge_index[1], (0, Ep - E), constant_values=N)
    ea = jnp.pad(edge_attr, ((0, Ep - E), (0, 0)))

    r2 = lambda v: v.reshape(1, -1)
    inv_bn = 1.0 / jnp.sqrt(jnp.float32(1.0 + 1e-5))

    h = _tc_linear(x, W_enc, r2(b_enc))
    ee0, eeh = _tc_edge_proj(ea, W_ein, r2(b_ein), W_eh, r2(b_eh))

    msg = _make_msg_kernel(N, Ep)
    layers = ((W1_0, b1_0, W2_0, b2_0, g_0, be_0),
              (W1_1, b1_1, W2_1, b2_1, g_1, be_1),
              (W1_2, b1_2, W2_2, b2_2, g_2, be_2))
    for li, (W1, b1, W2, b2, g, be) in enumerate(layers):
        ee = ee0 if li == 0 else eeh
        aggr2 = msg(h, ee, src, dst)
        h = _tc_layer_mlp(h, aggr2, W1, r2(b1), W2, r2(b2),
                          r2(g * inv_bn), r2(be))

    A, B = _tc_head_ab(h, Wm1[:D], r2(bm1), Wm1[D:])
    m = _make_edge_head_kernel(Ep)(A, B, src, dst)
    w2p = jnp.pad(Wm2, ((0, 0), (0, 8 - Wm2.shape[1])))
    b2p = jnp.pad(bm2, (0, 8 - bm2.shape[0]))
    out8 = _tc_linear(m, w2p, r2(b2p), block_r=2048)
    return out8[:E, :Wm2.shape[1]]


# speedup vs baseline: 1.9792x; 1.0008x over previous
import functools

import jax
import jax.numpy as jnp
from jax import lax
from jax.experimental import pallas as pl
from jax.experimental.pallas import tpu as pltpu
from jax.experimental.pallas import tpu_sc as plsc

F32 = jnp.float32

NC = 2
NS = 16
NW = NC * NS
LANES = 16
MSG_CHUNK = 40
HEAD_CHUNK = 80
NBUF = 4


def _cdiv(a, b):
    return (a + b - 1) // b


def _linear_body(x_ref, w_ref, b_ref, o_ref, *, act):
    y = jnp.dot(x_ref[...], w_ref[...], preferred_element_type=F32) + b_ref[...]
    if act:
        y = jnp.maximum(y, 0.0)
    o_ref[...] = y


def _tc_linear(x, w, b2d, act=False, block_r=None):
    R, K = x.shape
    Nn = w.shape[1]
    if block_r is None:
        block_r = 400 if R % 400 == 0 else 2048
    grid = (R // block_r,)
    return pl.pallas_call(
        functools.partial(_linear_body, act=act),
        grid=grid,
        in_specs=[
            pl.BlockSpec((block_r, K), lambda i: (i, 0)),
            pl.BlockSpec((K, Nn), lambda i: (0, 0)),
            pl.BlockSpec((1, Nn), lambda i: (0, 0)),
        ],
        out_specs=pl.BlockSpec((block_r, Nn), lambda i: (i, 0)),
        out_shape=jax.ShapeDtypeStruct((R, Nn), F32),
    )(x, w, b2d)


def _edge_proj_body(ea_ref, wi_ref, bi_ref, wh_ref, bh_ref, o0_ref, oh_ref):
    ea = ea_ref[...]
    o0_ref[...] = jnp.dot(ea, wi_ref[...], preferred_element_type=F32) + bi_ref[...]
    oh_ref[...] = jnp.dot(ea, wh_ref[...], preferred_element_type=F32) + bh_ref[...]


def _tc_edge_proj(ea, w_in, b_in, w_h, b_h):
    Ep, ED = ea.shape
    D = w_in.shape[1]
    block_r = 2048
    grid = (Ep // block_r,)
    return pl.pallas_call(
        _edge_proj_body,
        grid=grid,
        in_specs=[
            pl.BlockSpec((block_r, ED), lambda i: (i, 0)),
            pl.BlockSpec((ED, D), lambda i: (0, 0)),
            pl.BlockSpec((1, D), lambda i: (0, 0)),
            pl.BlockSpec((ED, D), lambda i: (0, 0)),
            pl.BlockSpec((1, D), lambda i: (0, 0)),
        ],
        out_specs=[
            pl.BlockSpec((block_r, D), lambda i: (i, 0)),
            pl.BlockSpec((block_r, D), lambda i: (i, 0)),
        ],
        out_shape=[
            jax.ShapeDtypeStruct((Ep, D), F32),
            jax.ShapeDtypeStruct((Ep, D), F32),
        ],
    )(ea, w_in, b_in, w_h, b_h)


def _mlp_body(h_ref, a_ref, w1_ref, b1_ref, w2_ref, b2_ref, sc_ref, be_ref, o_ref):
    z = h_ref[...] + a_ref[0] + a_ref[1]
    t = jnp.maximum(jnp.dot(z, w1_ref[...], preferred_element_type=F32) + b1_ref[...], 0.0)
    u = jnp.dot(t, w2_ref[...], preferred_element_type=F32) + b2_ref[...]
    o_ref[...] = jnp.maximum(u * sc_ref[...] + be_ref[...], 0.0)


def _tc_layer_mlp(h, aggr2, w1, b1, w2, b2, scale, be):
    R, D = h.shape
    H = w1.shape[1]
    block_r = 400
    grid = (R // block_r,)
    return pl.pallas_call(
        _mlp_body,
        grid=grid,
        in_specs=[
            pl.BlockSpec((block_r, D), lambda i: (i, 0)),
            pl.BlockSpec((2, block_r, D), lambda i: (0, i, 0)),
            pl.BlockSpec((D, H), lambda i: (0, 0)),
            pl.BlockSpec((1, H), lambda i: (0, 0)),
            pl.BlockSpec((H, H), lambda i: (0, 0)),
            pl.BlockSpec((1, H), lambda i: (0, 0)),
            pl.BlockSpec((1, H), lambda i: (0, 0)),
            pl.BlockSpec((1, H), lambda i: (0, 0)),
        ],
        out_specs=pl.BlockSpec((block_r, H), lambda i: (i, 0)),
        out_shape=jax.ShapeDtypeStruct((R, H), F32),
    )(h, aggr2, w1, b1, w2, b2, scale, be)


def _ab_body(h_ref, wt_ref, bt_ref, wb_ref, a_ref, b_ref):
    h = h_ref[...]
    a_ref[...] = jnp.dot(h, wt_ref[...], preferred_element_type=F32) + bt_ref[...]
    b_ref[...] = jnp.dot(h, wb_ref[...], preferred_element_type=F32)


def _tc_head_ab(h, w_top, bm1, w_bot):
    R, H = h.shape
    block_r = 400
    grid = (R // block_r,)
    return pl.pallas_call(
        _ab_body,
        grid=grid,
        in_specs=[
            pl.BlockSpec((block_r, H), lambda i: (i, 0)),
            pl.BlockSpec((H, H), lambda i: (0, 0)),
            pl.BlockSpec((1, H), lambda i: (0, 0)),
            pl.BlockSpec((H, H), lambda i: (0, 0)),
        ],
        out_specs=[
            pl.BlockSpec((block_r, H), lambda i: (i, 0)),
            pl.BlockSpec((block_r, H), lambda i: (i, 0)),
        ],
        out_shape=[
            jax.ShapeDtypeStruct((R, H), F32),
            jax.ShapeDtypeStruct((R, H), F32),
        ],
    )(h, w_top, bm1, w_bot)


def _chunk_sizes(total, step):
    sizes = []
    while total > 0:
        sizes.append(min(step, total))
        total -= sizes[-1]
    return sizes


@functools.lru_cache(maxsize=None)
def _make_msg_kernel(n_nodes, e_pad):
    per_w = e_pad // NW
    n_chunks = per_w // MSG_CHUNK
    n_outer = n_chunks // NBUF
    n_acc = _cdiv(n_nodes + 1, NS * 8) * NS * 8
    rows_t = n_acc // NS
    mesh = plsc.VectorSubcoreMesh(core_axis_name="c", subcore_axis_name="s",
                                  num_cores=NC, num_subcores=NS)

    NIDX = 2 * NBUF

    @functools.partial(
        pl.kernel,
        out_type=jax.ShapeDtypeStruct((NC, n_acc, 128), F32),
        mesh=mesh,
        scratch_types=[
            pltpu.VMEM((NIDX, MSG_CHUNK), jnp.int32),
            pltpu.VMEM((NIDX, MSG_CHUNK), jnp.int32),
            pltpu.VMEM((NBUF, MSG_CHUNK, 128), F32),
            pltpu.VMEM((NBUF, MSG_CHUNK, 128), F32),
            pltpu.VMEM((MSG_CHUNK, 128), F32),
            pltpu.VMEM_SHARED((n_acc, 128), F32),
            pltpu.SemaphoreType.DMA,
            [pltpu.SemaphoreType.DMA] * NBUF,
            [pltpu.SemaphoreType.DMA] * NBUF,
            [pltpu.SemaphoreType.DMA] * 2,
        ],
    )
    def msg_kernel(h_hbm, ee_hbm, src_hbm, dst_hbm, out_hbm,
                   sidx, didx, hbuf, eebuf, zbuf, acc,
                   sem_i, sem_g, sem_e, sem_s):
        c = lax.axis_index("c")
        s = lax.axis_index("s")
        wid = s * NC + c
        zero16 = jnp.zeros((LANES,), F32)

        def zrow(i, carry):
            for j in range(8):
                zbuf[i, pl.ds(j * LANES, LANES)] = zero16
            return carry

        lax.fori_loop(0, MSG_CHUNK, zrow, 0)
        zbase = pl.multiple_of(s * rows_t, 8)
        off = 0
        for sz in _chunk_sizes(rows_t, MSG_CHUNK):
            pltpu.sync_copy(zbuf.at[pl.ds(0, sz)], acc.at[pl.ds(zbase + off, sz)])
            off += sz
        plsc.subcore_barrier()

        ebase = wid * per_w

        def fire_loads(g, s8, s4):
            b = pl.multiple_of(ebase + g * MSG_CHUNK, 8)
            pltpu.async_copy(src_hbm.at[pl.ds(b, MSG_CHUNK)], sidx.at[s8], sem_i)
            pltpu.async_copy(dst_hbm.at[pl.ds(b, MSG_CHUNK)], didx.at[s8], sem_i)
            pltpu.async_copy(ee_hbm.at[pl.ds(b, MSG_CHUNK)], eebuf.at[s4],
                             sem_e[s4])

        def wait_idx(s8):
            pltpu.make_async_copy(src_hbm.at[pl.ds(0, MSG_CHUNK)], sidx.at[s8],
                                  sem_i).wait()
            pltpu.make_async_copy(dst_hbm.at[pl.ds(0, MSG_CHUNK)], didx.at[s8],
                                  sem_i).wait()

        def fire_gather(s8, s4):
            pltpu.async_copy(h_hbm.at[sidx.at[s8]], hbuf.at[s4], sem_g[s4])

        def wait_gather(s8, s4):
            pltpu.make_async_copy(h_hbm.at[sidx.at[s8]], hbuf.at[s4],
                                  sem_g[s4]).wait()

        def wait_ee(s4):
            pltpu.make_async_copy(ee_hbm.at[pl.ds(0, MSG_CHUNK)], eebuf.at[s4],
                                  sem_e[s4]).wait()

        def fire_scatter(s8, s4):
            pltpu.async_copy(hbuf.at[s4], acc.at[didx.at[s8]], sem_s[s4 % 2],
                             add=True)

        def wait_scatter(s8, s4):
            pltpu.make_async_copy(hbuf.at[s4], acc.at[didx.at[s8]],
                                  sem_s[s4 % 2]).wait()

        def compute(s4):
            def row(i, rcarry):
                for j in range(8):
                    ds16 = pl.ds(j * LANES, LANES)
                    hbuf[s4, i, ds16] = jnp.maximum(
                        hbuf[s4, i, ds16] + eebuf[s4, i, ds16], 0.0)
                return rcarry

            lax.fori_loop(0, MSG_CHUNK, row, 0)

        fire_loads(0, 0, 0)
        fire_loads(1, 1, 1)
        fire_loads(2, 2, 2)
        wait_idx(0)
        fire_gather(0, 0)
        wait_idx(1)
        fire_gather(1, 1)

        def outer(o, carry):
            for j in range(NBUF):
                g = o * NBUF + j
                s4 = j
                g8 = lax.rem(g, NIDX)
                g8_p2 = lax.rem(g + 2, NIDX)
                g8_p3 = lax.rem(g + 3, NIDX)
                g8_m2 = lax.rem(g + NIDX - 2, NIDX)

                def drain():
                    wait_scatter(g8_m2, (j + 2) % NBUF)

                if j >= 2:
                    drain()
                else:
                    pl.when(o >= 1)(drain)

                def gather_ahead():
                    wait_idx(g8_p2)
                    fire_gather(g8_p2, (j + 2) % NBUF)

                if j < NBUF - 2:
                    gather_ahead()
                else:
                    pl.when(o < n_outer - 1)(gather_ahead)

                def load_ahead():
                    fire_loads(g + 3, g8_p3, (j + 3) % NBUF)

                if j < NBUF - 3:
                    load_ahead()
                else:
                    pl.when(o < n_outer - 1)(load_ahead)

                wait_gather(g8, s4)
                wait_ee(s4)
                compute(s4)
                fire_scatter(g8, s4)
            return carry

        lax.fori_loop(0, n_outer, outer, 0)
        wait_scatter((n_chunks - 2) % NIDX, (n_chunks - 2) % NBUF)
        wait_scatter((n_chunks - 1) % NIDX, (n_chunks - 1) % NBUF)
        plsc.subcore_barrier()

        obase = pl.multiple_of(s * rows_t, 8)
        off = 0
        for sz in _chunk_sizes(rows_t, MSG_CHUNK):
            pltpu.sync_copy(acc.at[pl.ds(obase + off, sz)],
                            out_hbm.at[c, pl.ds(obase + off, sz)])
            off += sz

    return msg_kernel


@functools.lru_cache(maxsize=None)
def _make_edge_head_kernel(e_pad):
    per_w = e_pad // NW
    n_chunks = per_w // HEAD_CHUNK
    n_outer = n_chunks // NBUF
    mesh = plsc.VectorSubcoreMesh(core_axis_name="c", subcore_axis_name="s",
                                  num_cores=NC, num_subcores=NS)

    @functools.partial(
        pl.kernel,
        out_type=jax.ShapeDtypeStruct((e_pad, 128), F32),
        mesh=mesh,
        scratch_types=[
            pltpu.VMEM((NBUF, HEAD_CHUNK), jnp.int32),
            pltpu.VMEM((NBUF, HEAD_CHUNK), jnp.int32),
            pltpu.VMEM((NBUF, HEAD_CHUNK, 128), F32),
            pltpu.VMEM((NBUF, HEAD_CHUNK, 128), F32),
            pltpu.SemaphoreType.DMA,
            pltpu.SemaphoreType.DMA,
            pltpu.SemaphoreType.DMA,
            pltpu.SemaphoreType.DMA,
            pltpu.SemaphoreType.DMA,
            pltpu.SemaphoreType.DMA,
            pltpu.SemaphoreType.DMA,
        ],
    )
    def edge_head(a_hbm, b_hbm, src_hbm, dst_hbm, m_hbm,
                  sidx, didx, abuf, bbuf,
                  sem_i, sem_a0, sem_a1, sem_b0, sem_b1, sem_w0, sem_w1):
        c = lax.axis_index("c")
        s = lax.axis_index("s")
        wid = s * NC + c
        ebase = wid * per_w
        sem_a = (sem_a0, sem_a1)
        sem_b = (sem_b0, sem_b1)
        sem_w = (sem_w0, sem_w1)

        def fire_loads(g, sl):
            b = pl.multiple_of(ebase + g * HEAD_CHUNK, 8)
            pltpu.async_copy(src_hbm.at[pl.ds(b, HEAD_CHUNK)], sidx.at[sl], sem_i)
            pltpu.async_copy(dst_hbm.at[pl.ds(b, HEAD_CHUNK)], didx.at[sl], sem_i)

        def wait_idx(sl):
            pltpu.make_async_copy(src_hbm.at[pl.ds(0, HEAD_CHUNK)], sidx.at[sl],
                                  sem_i).wait()
            pltpu.make_async_copy(dst_hbm.at[pl.ds(0, HEAD_CHUNK)], didx.at[sl],
                                  sem_i).wait()

        def fire_gathers(sl):
            pltpu.async_copy(a_hbm.at[sidx.at[sl]], abuf.at[sl], sem_a[sl % 2])
            pltpu.async_copy(b_hbm.at[didx.at[sl]], bbuf.at[sl], sem_b[sl % 2])

        def wait_gathers(sl):
            pltpu.make_async_copy(a_hbm.at[sidx.at[sl]], abuf.at[sl],
                                  sem_a[sl % 2]).wait()
            pltpu.make_async_copy(b_hbm.at[didx.at[sl]], bbuf.at[sl],
                                  sem_b[sl % 2]).wait()

        def fire_write(g, sl):
            b = pl.multiple_of(ebase + g * HEAD_CHUNK, 8)
            pltpu.sync_copy(abuf.at[sl], m_hbm.at[pl.ds(b, HEAD_CHUNK)])

        def wait_write(sl):
            del sl

        def compute(sl):
            def row(i, rcarry):
                for j in range(8):
                    ds16 = pl.ds(j * LANES, LANES)
                    abuf[sl, i, ds16] = jnp.maximum(
                        abuf[sl, i, ds16] + bbuf[sl, i, ds16], 0.0)
                return rcarry

            lax.fori_loop(0, HEAD_CHUNK, row, 0)

        fire_loads(0, 0)
        wait_idx(0)
        fire_gathers(0)
        fire_loads(1, 1)

        def outer(o, carry):
            for j in range(NBUF):
                g = o * NBUF + j
                s1 = (j + 1) % NBUF
                s2 = (j + 2) % NBUF

                def gather_next(sl=s1):
                    wait_idx(sl)
                    fire_gathers(sl)

                if j < NBUF - 1:
                    gather_next()
                else:
                    pl.when(o < n_outer - 1)(gather_next)

                wait_gathers(j)
                compute(j)

                def drain(sl=s2):
                    wait_write(sl)

                if j >= 2:
                    drain()
                else:
                    pl.when(o >= 1)(drain)
                fire_write(g, j)

                def load_next(sl=s2, gg=g + 2):
                    fire_loads(gg, sl)

                if j < NBUF - 2:
                    load_next()
                else:
                    pl.when(o < n_outer - 1)(load_next)
            return carry

        lax.fori_loop(0, n_outer, outer, 0)
        wait_write(NBUF - 2)
        wait_write(NBUF - 1)

    return edge_head


def kernel(x, edge_index, edge_attr, W_enc, b_enc, W_ein, b_ein, W_eh, b_eh,
           W1_0, b1_0, W2_0, b2_0, g_0, be_0, W1_1, b1_1, W2_1, b2_1, g_1, be_1,
           W1_2, b1_2, W2_2, b2_2, g_2, be_2, Wm1, bm1, Wm2, bm2):
    N, D = x.shape
    E = edge_attr.shape[0]
    Ep = _cdiv(E, NW * HEAD_CHUNK * NBUF) * (NW * HEAD_CHUNK * NBUF)

    src = jnp.pad(edge_index[0], (0, Ep - E))
    dst = jnp.pad(edge_index[1], (0, Ep - E), constant_values=N)
    ea = jnp.pad(edge_attr, ((0, Ep - E), (0, 0)))

    r2 = lambda v: v.reshape(1, -1)
    inv_bn = 1.0 / jnp.sqrt(jnp.float32(1.0 + 1e-5))

    h = _tc_linear(x, W_enc, r2(b_enc))
    ee0, eeh = _tc_edge_proj(ea, W_ein, r2(b_ein), W_eh, r2(b_eh))

    msg = _make_msg_kernel(N, Ep)
    layers = ((W1_0, b1_0, W2_0, b2_0, g_0, be_0),
              (W1_1, b1_1, W2_1, b2_1, g_1, be_1),
              (W1_2, b1_2, W2_2, b2_2, g_2, be_2))
    for li, (W1, b1, W2, b2, g, be) in enumerate(layers):
        ee = ee0 if li == 0 else eeh
        aggr2 = msg(h, ee, src, dst)
        h = _tc_layer_mlp(h, aggr2, W1, r2(b1), W2, r2(b2),
                          r2(g * inv_bn), r2(be))

    A, B = _tc_head_ab(h, Wm1[:D], r2(bm1), Wm1[D:])
    m = _make_edge_head_kernel(Ep)(A, B, src, dst)
    w2p = jnp.pad(Wm2, ((0, 0), (0, 8 - Wm2.shape[1])))
    b2p = jnp.pad(bm2, (0, 8 - bm2.shape[0]))
    out8 = _tc_linear(m, w2p, r2(b2p), block_r=2048)
    return out8[:E, :Wm2.shape[1]]
